# baseline (speedup 1.0000x reference)
import jax
import jax.numpy as jnp
from jax import lax
from jax.experimental import pallas as pl
from jax.experimental.pallas import tpu as pltpu
from jax.experimental.pallas import tpu_sc as plsc

N_ROWS = 32768
D = 1024
N_BAGS = 16
L = 16
HALF = D // 2
R = 64
JGROUPS = HALF // L
MAXCH = 64
SPLIT = 16384
BLK = 2048


def _tree_sum(vals):
    while len(vals) > 1:
        vals = [vals[i] + vals[i + 1] for i in range(0, len(vals) - 1, 2)] + (
            [vals[-1]] if len(vals) % 2 else [])
    return vals[0]


def _sc_body(samples_hbm, csum_hbm, out_hbm, csum_v,
             tbl, buf0, buf1, buf2, acc16, outrow, shared, sem0, sem1, sem2):
    c = lax.axis_index("c")
    s = lax.axis_index("s")
    col0 = c * HALF
    bufs = (buf0, buf1, buf2)
    sems = (sem0, sem1, sem2)

    pltpu.sync_copy(csum_hbm, csum_v)
    csum_vec = csum_v[...]
    total = plsc.load_gather(csum_v, [jnp.full((L,), N_BAGS - 1, jnp.int32)])[0]

    lo0 = jnp.minimum(SPLIT, total)
    span = total - lo0
    q = ((span + (N_BAGS - 1)) // N_BAGS + 7) // 8 * 8
    w_lo = lo0 + jnp.minimum(s * q, span)
    w_hi = lo0 + jnp.minimum((s + 1) * q, span)

    def bag_of(row):
        le = csum_vec <= jnp.full((L,), row, jnp.int32)
        return plsc.all_reduce_population_count(le)[0]

    def csum_at(b):
        return plsc.load_gather(csum_v, [jnp.full((L,), b, jnp.int32)])[0]

    zero_row = jnp.zeros((L,), jnp.float32)

    def zrow(b, _):
        for j in range(JGROUPS):
            acc16[b, pl.ds(L * j, L)] = zero_row
        return 0

    lax.fori_loop(0, N_BAGS, zrow, 0)

    def seg_cond(state):
        r, b, n = state
        return r < w_hi

    def seg_body(state):
        r, b, n = state
        seg_end = jnp.minimum(csum_at(b), w_hi)
        abase = (r // 8) * 8

        def ch_cond(st):
            g, n2 = st
            return abase + g * R < seg_end

        def ch_body(st):
            g, n2 = st
            cbase = abase + g * R
            base = jnp.minimum(cbase, N_ROWS - R)
            tbl[0, n2] = base
            tbl[1, n2] = jnp.maximum(r, cbase) - base
            tbl[2, n2] = jnp.minimum(seg_end, cbase + R) - base
            tbl[3, n2] = b
            return g + 1, n2 + 1

        _, n = lax.while_loop(ch_cond, ch_body, (0, n))
        return seg_end, b + 1, n

    b_init = bag_of(w_lo)
    _, _, n_chunks = lax.while_loop(seg_cond, seg_body, (w_lo, b_init, 0))

    def start_dma(k, b):
        base = pl.multiple_of(tbl[0, k], 8)
        pltpu.async_copy(
            samples_hbm.at[pl.ds(base, R), pl.ds(col0, HALF)],
            bufs[b], sems[b])

    def wait_dma(b):
        pltpu.make_async_copy(
            samples_hbm.at[pl.ds(0, R), pl.ds(col0, HALF)],
            bufs[b], sems[b]).wait()

    def compute(k, b):
        buf = bufs[b]
        lo = tbl[1, k]
        hi = tbl[2, k]
        bag = tbl[3, k]

        def zero_one(r, _):
            for j in range(JGROUPS):
                buf[r, pl.ds(L * j, L)] = zero_row
            return 0

        lax.fori_loop(0, lo, zero_one, 0)
        lax.fori_loop(hi, R, zero_one, 0)

        @plsc.parallel_loop(0, JGROUPS)
        def jstep(j):
            off = pl.ds(L * j, L)
            acc16[bag, off] = acc16[bag, off] + _tree_sum(
                [buf[r, off] for r in range(R)])

    NBUF = 3
    for p in range(NBUF - 1):
        @pl.when(p < n_chunks)
        def _(p=p):
            start_dma(p, p)

    def ring_body(i, _):
        k3 = i * NBUF
        for b in range(NBUF):
            k = k3 + b

            @pl.when(k < n_chunks)
            def _():
                wait_dma(b)

                @pl.when(k + (NBUF - 1) < n_chunks)
                def _():
                    start_dma(k + (NBUF - 1), (b + NBUF - 1) % NBUF)

                compute(k, b)
        return 0

    lax.fori_loop(0, (n_chunks + NBUF - 1) // NBUF, ring_body, 0)

    pltpu.sync_copy(acc16, shared.at[s])
    plsc.subcore_barrier()
    pltpu.sync_copy(shared.at[:, s], acc16)
    for j in range(JGROUPS):
        off = pl.ds(L * j, L)
        outrow[off] = _tree_sum([acc16[t, off] for t in range(N_BAGS)])
    out_off = pl.multiple_of(s * D + col0, HALF)
    pltpu.sync_copy(outrow, out_hbm.at[pl.ds(out_off, HALF)])


def _sc_partial(samples, csum):
    mesh = plsc.VectorSubcoreMesh(core_axis_name="c", subcore_axis_name="s")
    run = pl.kernel(
        _sc_body,
        out_type=jax.ShapeDtypeStruct((N_BAGS * D,), jnp.float32),
        mesh=mesh,
        compiler_params=pltpu.CompilerParams(needs_layout_passes=False),
        scratch_types=[
            pltpu.VMEM((L,), jnp.int32),
            pltpu.SMEM((4, MAXCH), jnp.int32),
            pltpu.VMEM((R, HALF), jnp.float32),
            pltpu.VMEM((R, HALF), jnp.float32),
            pltpu.VMEM((R, HALF), jnp.float32),
            pltpu.VMEM((N_BAGS, HALF), jnp.float32),
            pltpu.VMEM((HALF,), jnp.float32),
            pltpu.VMEM_SHARED((16, N_BAGS, HALF), jnp.float32),
            pltpu.SemaphoreType.DMA,
            pltpu.SemaphoreType.DMA,
            pltpu.SemaphoreType.DMA,
        ],
    )
    return run(samples, csum).reshape(N_BAGS, D)


def _tc_body(starts_ref, ends_ref, x_ref, out_ref):
    i = pl.program_id(0)

    @pl.when(i == 0)
    def _():
        out_ref[...] = jnp.zeros_like(out_ref)

    rg = lax.broadcasted_iota(jnp.int32, (N_BAGS, BLK), 1) + i * BLK
    starts = starts_ref[0, :][:, None]
    ends = ends_ref[0, :][:, None]
    m = ((rg >= starts) & (rg < ends)).astype(jnp.float32)
    out_ref[...] += jnp.dot(m, x_ref[...],
                            preferred_element_type=jnp.float32)


def _tc_partial(samples, starts2d, ends2d):
    return pl.pallas_call(
        _tc_body,
        grid=(SPLIT // BLK,),
        in_specs=[
            pl.BlockSpec((1, N_BAGS), lambda i: (0, 0)),
            pl.BlockSpec((1, N_BAGS), lambda i: (0, 0)),
            pl.BlockSpec((BLK, D), lambda i: (i, 0)),
        ],
        out_specs=pl.BlockSpec((N_BAGS, D), lambda i: (0, 0)),
        out_shape=jax.ShapeDtypeStruct((N_BAGS, D), jnp.float32),
        compiler_params=pltpu.CompilerParams(
            dimension_semantics=("arbitrary",)),
    )(starts2d, ends2d, samples)


def _combine_body(a_ref, b_ref, cnt_ref, out_ref):
    out_ref[...] = (a_ref[...] + b_ref[...]) / cnt_ref[...]


def _combine(ptc, psc, cnt_col):
    return pl.pallas_call(
        _combine_body,
        out_shape=jax.ShapeDtypeStruct((N_BAGS, D), jnp.float32),
    )(ptc, psc, cnt_col)


@jax.jit
def kernel(samples, bags_num_samples):
    csum = jnp.cumsum(bags_num_samples)
    starts2d = (csum - bags_num_samples).reshape(1, N_BAGS)
    ends2d = jnp.minimum(csum, SPLIT).reshape(1, N_BAGS)
    psc = _sc_partial(samples, csum)
    ptc = _tc_partial(samples, starts2d, ends2d)
    cnt_col = bags_num_samples.astype(jnp.float32).reshape(N_BAGS, 1)
    return _combine(ptc, psc, cnt_col)


# speedup vs baseline: 1.0062x; 1.0062x over previous
import jax
import jax.numpy as jnp
from jax import lax
from jax.experimental import pallas as pl
from jax.experimental.pallas import tpu as pltpu
from jax.experimental.pallas import tpu_sc as plsc

N_ROWS = 32768
D = 1024
N_BAGS = 16
L = 16
HALF = D // 2
R = 64
JGROUPS = HALF // L
MAXCH = 64
SPLIT = 15360
BLK = 2048


def _tree_sum(vals):
    while len(vals) > 1:
        vals = [vals[i] + vals[i + 1] for i in range(0, len(vals) - 1, 2)] + (
            [vals[-1]] if len(vals) % 2 else [])
    return vals[0]


def _sc_body(samples_hbm, csum_hbm, out_hbm, csum_v,
             tbl, buf0, buf1, buf2, acc16, outrow, shared, sem0, sem1, sem2):
    c = lax.axis_index("c")
    s = lax.axis_index("s")
    col0 = c * HALF
    bufs = (buf0, buf1, buf2)
    sems = (sem0, sem1, sem2)

    pltpu.sync_copy(csum_hbm, csum_v)
    csum_vec = csum_v[...]
    total = plsc.load_gather(csum_v, [jnp.full((L,), N_BAGS - 1, jnp.int32)])[0]

    lo0 = jnp.minimum(SPLIT, total)
    span = total - lo0
    q = ((span + (N_BAGS - 1)) // N_BAGS + 7) // 8 * 8
    w_lo = lo0 + jnp.minimum(s * q, span)
    w_hi = lo0 + jnp.minimum((s + 1) * q, span)

    def bag_of(row):
        le = csum_vec <= jnp.full((L,), row, jnp.int32)
        return plsc.all_reduce_population_count(le)[0]

    def csum_at(b):
        return plsc.load_gather(csum_v, [jnp.full((L,), b, jnp.int32)])[0]

    zero_row = jnp.zeros((L,), jnp.float32)

    def zrow(b, _):
        for j in range(JGROUPS):
            acc16[b, pl.ds(L * j, L)] = zero_row
        return 0

    lax.fori_loop(0, N_BAGS, zrow, 0)

    def seg_cond(state):
        r, b, n = state
        return r < w_hi

    def seg_body(state):
        r, b, n = state
        seg_end = jnp.minimum(csum_at(b), w_hi)
        abase = (r // 8) * 8

        def ch_cond(st):
            g, n2 = st
            return abase + g * R < seg_end

        def ch_body(st):
            g, n2 = st
            cbase = abase + g * R
            base = jnp.minimum(cbase, N_ROWS - R)
            tbl[0, n2] = base
            tbl[1, n2] = jnp.maximum(r, cbase) - base
            tbl[2, n2] = jnp.minimum(seg_end, cbase + R) - base
            tbl[3, n2] = b
            return g + 1, n2 + 1

        _, n = lax.while_loop(ch_cond, ch_body, (0, n))
        return seg_end, b + 1, n

    b_init = bag_of(w_lo)
    _, _, n_chunks = lax.while_loop(seg_cond, seg_body, (w_lo, b_init, 0))

    def start_dma(k, b):
        base = pl.multiple_of(tbl[0, k], 8)
        pltpu.async_copy(
            samples_hbm.at[pl.ds(base, R), pl.ds(col0, HALF)],
            bufs[b], sems[b])

    def wait_dma(b):
        pltpu.make_async_copy(
            samples_hbm.at[pl.ds(0, R), pl.ds(col0, HALF)],
            bufs[b], sems[b]).wait()

    def compute(k, b):
        buf = bufs[b]
        lo = tbl[1, k]
        hi = tbl[2, k]
        bag = tbl[3, k]

        def zero_one(r, _):
            for j in range(JGROUPS):
                buf[r, pl.ds(L * j, L)] = zero_row
            return 0

        lax.fori_loop(0, lo, zero_one, 0)
        lax.fori_loop(hi, R, zero_one, 0)

        @plsc.parallel_loop(0, JGROUPS)
        def jstep(j):
            off = pl.ds(L * j, L)
            acc16[bag, off] = acc16[bag, off] + _tree_sum(
                [buf[r, off] for r in range(R)])

    NBUF = 3
    for p in range(NBUF - 1):
        @pl.when(p < n_chunks)
        def _(p=p):
            start_dma(p, p)

    def ring_body(i, _):
        k3 = i * NBUF
        for b in range(NBUF):
            k = k3 + b

            @pl.when(k < n_chunks)
            def _():
                wait_dma(b)

                @pl.when(k + (NBUF - 1) < n_chunks)
                def _():
                    start_dma(k + (NBUF - 1), (b + NBUF - 1) % NBUF)

                compute(k, b)
        return 0

    lax.fori_loop(0, (n_chunks + NBUF - 1) // NBUF, ring_body, 0)

    pltpu.sync_copy(acc16, shared.at[s])
    plsc.subcore_barrier()
    pltpu.sync_copy(shared.at[:, s], acc16)
    for j in range(JGROUPS):
        off = pl.ds(L * j, L)
        outrow[off] = _tree_sum([acc16[t, off] for t in range(N_BAGS)])
    out_off = pl.multiple_of(s * D + col0, HALF)
    pltpu.sync_copy(outrow, out_hbm.at[pl.ds(out_off, HALF)])


def _sc_partial(samples, csum):
    mesh = plsc.VectorSubcoreMesh(core_axis_name="c", subcore_axis_name="s")
    run = pl.kernel(
        _sc_body,
        out_type=jax.ShapeDtypeStruct((N_BAGS * D,), jnp.float32),
        mesh=mesh,
        compiler_params=pltpu.CompilerParams(needs_layout_passes=False),
        scratch_types=[
            pltpu.VMEM((L,), jnp.int32),
            pltpu.SMEM((4, MAXCH), jnp.int32),
            pltpu.VMEM((R, HALF), jnp.float32),
            pltpu.VMEM((R, HALF), jnp.float32),
            pltpu.VMEM((R, HALF), jnp.float32),
            pltpu.VMEM((N_BAGS, HALF), jnp.float32),
            pltpu.VMEM((HALF,), jnp.float32),
            pltpu.VMEM_SHARED((16, N_BAGS, HALF), jnp.float32),
            pltpu.SemaphoreType.DMA,
            pltpu.SemaphoreType.DMA,
            pltpu.SemaphoreType.DMA,
        ],
    )
    return run(samples, csum).reshape(N_BAGS, D)


def _tc_body(starts_ref, ends_ref, x_ref, out_ref):
    i = pl.program_id(0)

    @pl.when(i == 0)
    def _():
        out_ref[...] = jnp.zeros_like(out_ref)

    rg = lax.broadcasted_iota(jnp.int32, (N_BAGS, BLK), 1) + i * BLK
    starts = starts_ref[0, :][:, None]
    ends = ends_ref[0, :][:, None]
    m = ((rg >= starts) & (rg < ends)).astype(jnp.float32)
    out_ref[...] += jnp.dot(m, x_ref[...],
                            preferred_element_type=jnp.float32)


def _tc_partial(samples, starts2d, ends2d):
    return pl.pallas_call(
        _tc_body,
        grid=(SPLIT // BLK,),
        in_specs=[
            pl.BlockSpec((1, N_BAGS), lambda i: (0, 0)),
            pl.BlockSpec((1, N_BAGS), lambda i: (0, 0)),
            pl.BlockSpec((BLK, D), lambda i: (i, 0)),
        ],
        out_specs=pl.BlockSpec((N_BAGS, D), lambda i: (0, 0)),
        out_shape=jax.ShapeDtypeStruct((N_BAGS, D), jnp.float32),
        compiler_params=pltpu.CompilerParams(
            dimension_semantics=("arbitrary",)),
    )(starts2d, ends2d, samples)


def _combine_body(a_ref, b_ref, cnt_ref, out_ref):
    out_ref[...] = (a_ref[...] + b_ref[...]) / cnt_ref[...]


def _combine(ptc, psc, cnt_col):
    return pl.pallas_call(
        _combine_body,
        out_shape=jax.ShapeDtypeStruct((N_BAGS, D), jnp.float32),
    )(ptc, psc, cnt_col)


@jax.jit
def kernel(samples, bags_num_samples):
    csum = jnp.cumsum(bags_num_samples)
    starts2d = (csum - bags_num_samples).reshape(1, N_BAGS)
    ends2d = jnp.minimum(csum, SPLIT).reshape(1, N_BAGS)
    psc = _sc_partial(samples, csum)
    ptc = _tc_partial(samples, starts2d, ends2d)
    cnt_col = bags_num_samples.astype(jnp.float32).reshape(N_BAGS, 1)
    return _combine(ptc, psc, cnt_col)
